# baseline (speedup 1.0000x reference)
import jax
import jax.numpy as jnp
from jax import lax
from jax.experimental import pallas as pl
from jax.experimental.pallas import tpu as pltpu
from jax.experimental.pallas import tpu_sc as plsc

V = 100000
F = 200000
NNZ = 700000

NC = 2
NS = 16
NW = NC * NS
LANES = 16

S = 2736
NSUB = 8
Q = NSUB * S
OVERLAP = NW * Q - NNZ

VP = 100096
CHK = VP // NS


def _sc_spmv_call(vflat, cpat48, row_l, col_l, val_l, row_k, col_k, val_k):
    mesh = plsc.VectorSubcoreMesh(core_axis_name="c", subcore_axis_name="s",
                                  num_cores=NC, num_subcores=NS)

    def body(vflat_hbm, cpat_hbm, rL_hbm, cL_hbm, vL_hbm,
             rK_hbm, cK_hbm, vK_hbm,
             out_hbm, vc3_hbm,
             aLx, aLy, aLz, aKx, aKy, aKz, sv3,
             vbuf, xb, cp_v,
             col_v, row_v, val_v, col_w, row_w, val_w,
             ic0, ic1, ic2, jc0, jc1, jc2, gx, gy, gz, hx, hy, hz,
             sem0, sem1, sem2, lsem0, lsem1, ssem0, ssem1):
        cid = lax.axis_index("c")
        sid = lax.axis_index("s")

        def zero_body(i, _):
            xb[pl.ds(i * LANES, LANES)] = jnp.zeros((LANES,), jnp.float32)
            return 0
        lax.fori_loop(0, CHK // LANES, zero_body, 0)
        for acc in (aLx, aLy, aLz, aKx, aKy, aKz):
            pltpu.sync_copy(xb, acc.at[pl.ds(sid * CHK, CHK)])

        base0 = jnp.minimum(sid * CHK, V - CHK)
        pltpu.sync_copy(cpat_hbm, cp_v)
        q0 = cp_v[pl.ds(0, LANES)]
        q1 = cp_v[pl.ds(LANES, LANES)]
        q2 = cp_v[pl.ds(2 * LANES, LANES)]

        for off, ln in ((0, 9408), (9408, 9360)):
            pltpu.sync_copy(vflat_hbm.at[pl.ds(base0 * 3 + off, ln)],
                            vbuf.at[pl.ds(0, ln)])

            def cadd_body(g, _):
                b = g * (3 * LANES)
                vbuf[pl.ds(b, LANES)] = vbuf[pl.ds(b, LANES)] + q0
                vbuf[pl.ds(b + LANES, LANES)] = (
                    vbuf[pl.ds(b + LANES, LANES)] + q1)
                vbuf[pl.ds(b + 2 * LANES, LANES)] = (
                    vbuf[pl.ds(b + 2 * LANES, LANES)] + q2)
                return 0
            lax.fori_loop(0, ln // (3 * LANES), cadd_body, 0)
            pltpu.sync_copy(vbuf.at[pl.ds(0, ln)],
                            sv3.at[pl.ds(base0 * 3 + off, ln)])
            @pl.when(cid == 0)
            def _export_vc():
                pltpu.sync_copy(vbuf.at[pl.ds(0, ln)],
                                vc3_hbm.at[pl.ds(base0 * 3 + off, ln)])
        plsc.subcore_barrier()

        wid = cid * NS + sid
        is_last = wid == NW - 1

        chunks = []
        for m, (r_hbm, c_hbm, v_hbm, accs) in enumerate(
                ((rL_hbm, cL_hbm, vL_hbm, (aLx, aLy, aLz)),
                 (rK_hbm, cK_hbm, vK_hbm, (aKx, aKy, aKz)))):
            for j in range(NSUB):
                base = wid * Q + j * S
                if j == NSUB - 1:
                    base = jnp.where(is_last, NNZ - S, base)
                chunks.append((r_hbm, c_hbm, v_hbm, accs, base,
                               j == NSUB - 1))

        rows = (row_v, row_w)
        cols = (col_v, col_w)
        vals = (val_v, val_w)
        lsems = (lsem0, lsem1)
        ics = ((ic0, ic1, ic2), (jc0, jc1, jc2))
        gs = ((gx, gy, gz), (hx, hy, hz))
        ssems = (ssem0, ssem1)

        def fire_loads(k):
            r_hbm, c_hbm, v_hbm, _, base, _ = chunks[k]
            b = k % 2
            return (
                pltpu.async_copy(r_hbm.at[pl.ds(base, S)], rows[b], lsems[b]),
                pltpu.async_copy(c_hbm.at[pl.ds(base, S)], cols[b], lsems[b]),
                pltpu.async_copy(v_hbm.at[pl.ds(base, S)], vals[b], lsems[b]),
            )

        pending = fire_loads(0)
        pend_scat = [None, None]
        for k in range(len(chunks)):
            _, _, _, (ax, ay, az), _, tail = chunks[k]
            b = k % 2
            for d in pending:
                d.wait()
            row_b, col_b, val_b = rows[b], cols[b], vals[b]
            i0, i1, i2 = ics[b]
            g0, g1, g2 = gs[b]
            if tail:
                @pl.when(is_last)
                def _mask_tail():
                    def zv(i, _):
                        val_b[pl.ds(i * LANES, LANES)] = (
                            jnp.zeros((LANES,), jnp.float32))
                        return 0
                    lax.fori_loop(0, OVERLAP // LANES, zv, 0)

            def idx_body(i, _):
                sl = pl.ds(i * LANES, LANES)
                c3 = col_b[sl] * 3
                i0[sl] = c3
                i1[sl] = c3 + 1
                i2[sl] = c3 + 2
                return 0
            lax.fori_loop(0, S // LANES, idx_body, 0)
            if pend_scat[b] is not None:
                for d in pend_scat[b]:
                    d.wait()
                pend_scat[b] = None
            d0 = pltpu.async_copy(sv3.at[i0], g0, sem0)
            d1 = pltpu.async_copy(sv3.at[i1], g1, sem1)
            d2 = pltpu.async_copy(sv3.at[i2], g2, sem2)
            if k + 1 < len(chunks):
                if pend_scat[(k + 1) % 2] is not None:
                    for d in pend_scat[(k + 1) % 2]:
                        d.wait()
                    pend_scat[(k + 1) % 2] = None
                pending = fire_loads(k + 1)
            d0.wait()
            d1.wait()
            d2.wait()

            def mul_body(i, _):
                sl = pl.ds(i * LANES, LANES)
                w = val_b[sl]
                g0[sl] = g0[sl] * w
                g1[sl] = g1[sl] * w
                g2[sl] = g2[sl] * w
                return 0
            lax.fori_loop(0, S // LANES, mul_body, 0)

            pend_scat[b] = (
                pltpu.async_copy(g0, ax.at[row_b], ssems[b], add=True),
                pltpu.async_copy(g1, ay.at[row_b], ssems[b], add=True),
                pltpu.async_copy(g2, az.at[row_b], ssems[b], add=True),
            )

        for ds_ in pend_scat:
            if ds_ is not None:
                for d in ds_:
                    d.wait()
        plsc.subcore_barrier()

        for j, acc in enumerate((aLx, aLy, aLz, aKx, aKy, aKz)):
            off = (cid * 6 + j) * VP + sid * CHK
            pltpu.sync_copy(acc.at[pl.ds(sid * CHK, CHK)], xb)
            pltpu.sync_copy(xb, out_hbm.at[pl.ds(off, CHK)])

    kfn = pl.kernel(
        body,
        out_type=[jax.ShapeDtypeStruct((NC * 6 * VP,), jnp.float32),
                  jax.ShapeDtypeStruct((3 * V,), jnp.float32)],
        mesh=mesh,
        scratch_types=[
            pltpu.VMEM_SHARED((VP,), jnp.float32),
            pltpu.VMEM_SHARED((VP,), jnp.float32),
            pltpu.VMEM_SHARED((VP,), jnp.float32),
            pltpu.VMEM_SHARED((VP,), jnp.float32),
            pltpu.VMEM_SHARED((VP,), jnp.float32),
            pltpu.VMEM_SHARED((VP,), jnp.float32),
            pltpu.VMEM_SHARED((3 * VP,), jnp.float32),
            pltpu.VMEM((9408,), jnp.float32),
            pltpu.VMEM((CHK,), jnp.float32),
            pltpu.VMEM((3 * LANES,), jnp.float32),
            pltpu.VMEM((S,), jnp.int32),
            pltpu.VMEM((S,), jnp.int32),
            pltpu.VMEM((S,), jnp.float32),
            pltpu.VMEM((S,), jnp.int32),
            pltpu.VMEM((S,), jnp.int32),
            pltpu.VMEM((S,), jnp.float32),
            pltpu.VMEM((S,), jnp.int32),
            pltpu.VMEM((S,), jnp.int32),
            pltpu.VMEM((S,), jnp.int32),
            pltpu.VMEM((S,), jnp.int32),
            pltpu.VMEM((S,), jnp.int32),
            pltpu.VMEM((S,), jnp.int32),
            pltpu.VMEM((S,), jnp.float32),
            pltpu.VMEM((S,), jnp.float32),
            pltpu.VMEM((S,), jnp.float32),
            pltpu.VMEM((S,), jnp.float32),
            pltpu.VMEM((S,), jnp.float32),
            pltpu.VMEM((S,), jnp.float32),
            pltpu.SemaphoreType.DMA,
            pltpu.SemaphoreType.DMA,
            pltpu.SemaphoreType.DMA,
            pltpu.SemaphoreType.DMA,
            pltpu.SemaphoreType.DMA,
            pltpu.SemaphoreType.DMA,
            pltpu.SemaphoreType.DMA,
        ],
    )
    return kfn(vflat, cpat48, row_l, col_l, val_l, row_k, col_k, val_k)


def _tc_flatten_call(ind_l, ind_k):
    BI = 7168
    grid = pl.cdiv(NNZ, BI)

    def body(l_ref, k_ref, rl_ref, cl_ref, rk_ref, ck_ref):
        rl_ref[...] = l_ref[0]
        cl_ref[...] = l_ref[1]
        rk_ref[...] = k_ref[0]
        ck_ref[...] = k_ref[1]

    return pl.pallas_call(
        body,
        grid=(grid,),
        in_specs=[
            pl.BlockSpec((2, BI), lambda i: (0, i)),
            pl.BlockSpec((2, BI), lambda i: (0, i)),
        ],
        out_specs=[pl.BlockSpec((BI,), lambda i: (i,)) for _ in range(4)],
        out_shape=[jax.ShapeDtypeStruct((NNZ,), jnp.int32) for _ in range(4)],
    )(ind_l, ind_k)


def _tc_reduce_call(parts_flat):
    def body(p_ref, out_ref):
        def comp(j):
            return (p_ref[pl.ds(j * VP, VP)] +
                    p_ref[pl.ds((6 + j) * VP, VP)])
        eps = jnp.float32(1e-12)
        lx, ly, lz = comp(0) + eps, comp(1) + eps, comp(2) + eps
        norm = jnp.sqrt(lx * lx + ly * ly + lz * lz)
        kx, ky, kz = comp(3), comp(4), comp(5)
        ksq = kx * kx + ky * ky + kz * kz
        out_ref[0, 0] = jnp.sum(norm) / jnp.float32(V)
        out_ref[0, 1] = jnp.sum(ksq) / jnp.float32(V)

    return pl.pallas_call(
        body,
        out_specs=pl.BlockSpec(memory_space=pltpu.SMEM),
        out_shape=jax.ShapeDtypeStruct((1, 2), jnp.float32),
    )(parts_flat)


def kernel(vertices, center, faces, L_indices, L_values, K_indices, K_values,
           total_num):
    one_i = jnp.asarray(total_num, dtype=jnp.int32) // 4
    one_f = one_i.astype(jnp.float32)

    vflat = vertices.reshape(3 * V)
    c3 = center.reshape(3)
    cpat48 = jnp.tile(c3, LANES)

    row_l, col_l, row_k, col_k = _tc_flatten_call(L_indices, K_indices)
    parts, vc3 = _sc_spmv_call(vflat, cpat48, row_l, col_l, L_values,
                               row_k, col_k, K_values)
    losses = _tc_reduce_call(parts)

    vc = vc3.reshape(1, V, 3)
    verts_out = jnp.tile(vc, (4, 1, 1)) * one_f
    faces_out = jnp.tile(faces[None], (4, 1, 1)) * one_i

    laplacian_loss = losses[0, 0]
    hexagon_loss = losses[0, 1]
    zero = jnp.float32(0.0)
    return (verts_out, faces_out, laplacian_loss, hexagon_loss, zero, zero)


# speedup vs baseline: 1.1156x; 1.1156x over previous
import jax
import jax.numpy as jnp
from jax import lax
from jax.experimental import pallas as pl
from jax.experimental.pallas import tpu as pltpu
from jax.experimental.pallas import tpu_sc as plsc

V = 100000
F = 200000
NNZ = 700000

NC = 2
NS = 16
NW = NC * NS
LANES = 16

S = 2736
NSUB = 8
Q = NSUB * S
OVERLAP = NW * Q - NNZ

VP = 100096
CHK = VP // NS


def _sc_spmv_call(vflat, cpat48, ind_l, val_l, ind_k, val_k):
    mesh = plsc.VectorSubcoreMesh(core_axis_name="c", subcore_axis_name="s",
                                  num_cores=NC, num_subcores=NS)

    def body(vflat_hbm, cpat_hbm, iL_hbm, vL_hbm, iK_hbm, vK_hbm,
             out_hbm, vc3_hbm,
             aLx, aLy, aLz, aKx, aKy, aKz, sv3,
             vbuf, xb, cp_v,
             col_v, row_v, val_v, col_w, row_w, val_w,
             ic0, ic1, ic2, jc0, jc1, jc2, gx, gy, gz, hx, hy, hz,
             sem0, sem1, sem2, lsem0, lsem1, ssem0, ssem1):
        cid = lax.axis_index("c")
        sid = lax.axis_index("s")

        def zero_body(i, _):
            xb[pl.ds(i * LANES, LANES)] = jnp.zeros((LANES,), jnp.float32)
            return 0
        lax.fori_loop(0, CHK // LANES, zero_body, 0)
        for acc in (aLx, aLy, aLz, aKx, aKy, aKz):
            pltpu.sync_copy(xb, acc.at[pl.ds(sid * CHK, CHK)])

        base0 = jnp.minimum(sid * CHK, V - CHK)
        pltpu.sync_copy(cpat_hbm, cp_v)
        q0 = cp_v[pl.ds(0, LANES)]
        q1 = cp_v[pl.ds(LANES, LANES)]
        q2 = cp_v[pl.ds(2 * LANES, LANES)]

        for off, ln in ((0, 9408), (9408, 9360)):
            pltpu.sync_copy(vflat_hbm.at[pl.ds(base0 * 3 + off, ln)],
                            vbuf.at[pl.ds(0, ln)])

            def cadd_body(g, _):
                b = g * (3 * LANES)
                vbuf[pl.ds(b, LANES)] = vbuf[pl.ds(b, LANES)] + q0
                vbuf[pl.ds(b + LANES, LANES)] = (
                    vbuf[pl.ds(b + LANES, LANES)] + q1)
                vbuf[pl.ds(b + 2 * LANES, LANES)] = (
                    vbuf[pl.ds(b + 2 * LANES, LANES)] + q2)
                return 0
            lax.fori_loop(0, ln // (3 * LANES), cadd_body, 0)
            pltpu.sync_copy(vbuf.at[pl.ds(0, ln)],
                            sv3.at[pl.ds(base0 * 3 + off, ln)])
            @pl.when(cid == 0)
            def _export_vc():
                pltpu.sync_copy(vbuf.at[pl.ds(0, ln)],
                                vc3_hbm.at[pl.ds(base0 * 3 + off, ln)])
        plsc.subcore_barrier()

        wid = cid * NS + sid
        is_last = wid == NW - 1

        chunks = []
        for m, (i_hbm, v_hbm, accs) in enumerate(
                ((iL_hbm, vL_hbm, (aLx, aLy, aLz)),
                 (iK_hbm, vK_hbm, (aKx, aKy, aKz)))):
            for j in range(NSUB):
                base = wid * Q + j * S
                if j == NSUB - 1:
                    base = jnp.where(is_last, NNZ - S, base)
                chunks.append((i_hbm, v_hbm, accs, base, j == NSUB - 1))

        rows = (row_v, row_w)
        cols = (col_v, col_w)
        vals = (val_v, val_w)
        lsems = (lsem0, lsem1)
        ics = ((ic0, ic1, ic2), (jc0, jc1, jc2))
        gs = ((gx, gy, gz), (hx, hy, hz))
        ssems = (ssem0, ssem1)

        def fire_loads(k):
            i_hbm, v_hbm, _, base, _ = chunks[k]
            b = k % 2
            return (
                pltpu.async_copy(i_hbm.at[pl.ds(base, S)], rows[b], lsems[b]),
                pltpu.async_copy(i_hbm.at[pl.ds(NNZ + base, S)], cols[b],
                                 lsems[b]),
                pltpu.async_copy(v_hbm.at[pl.ds(base, S)], vals[b], lsems[b]),
            )

        pending = fire_loads(0)
        pend_scat = [None, None]
        for k in range(len(chunks)):
            _, _, (ax, ay, az), _, tail = chunks[k]
            b = k % 2
            for d in pending:
                d.wait()
            row_b, col_b, val_b = rows[b], cols[b], vals[b]
            i0, i1, i2 = ics[b]
            g0, g1, g2 = gs[b]
            if tail:
                @pl.when(is_last)
                def _mask_tail():
                    def zv(i, _):
                        val_b[pl.ds(i * LANES, LANES)] = (
                            jnp.zeros((LANES,), jnp.float32))
                        return 0
                    lax.fori_loop(0, OVERLAP // LANES, zv, 0)

            def idx_body(i, _):
                sl = pl.ds(i * LANES, LANES)
                c3 = col_b[sl] * 3
                i0[sl] = c3
                i1[sl] = c3 + 1
                i2[sl] = c3 + 2
                return 0
            lax.fori_loop(0, S // LANES, idx_body, 0)
            if pend_scat[b] is not None:
                for d in pend_scat[b]:
                    d.wait()
                pend_scat[b] = None
            d0 = pltpu.async_copy(sv3.at[i0], g0, sem0)
            d1 = pltpu.async_copy(sv3.at[i1], g1, sem1)
            d2 = pltpu.async_copy(sv3.at[i2], g2, sem2)
            if k + 1 < len(chunks):
                if pend_scat[(k + 1) % 2] is not None:
                    for d in pend_scat[(k + 1) % 2]:
                        d.wait()
                    pend_scat[(k + 1) % 2] = None
                pending = fire_loads(k + 1)
            d0.wait()
            d1.wait()
            d2.wait()

            def mul_body(i, _):
                sl = pl.ds(i * LANES, LANES)
                w = val_b[sl]
                g0[sl] = g0[sl] * w
                g1[sl] = g1[sl] * w
                g2[sl] = g2[sl] * w
                return 0
            lax.fori_loop(0, S // LANES, mul_body, 0)

            pend_scat[b] = (
                pltpu.async_copy(g0, ax.at[row_b], ssems[b], add=True),
                pltpu.async_copy(g1, ay.at[row_b], ssems[b], add=True),
                pltpu.async_copy(g2, az.at[row_b], ssems[b], add=True),
            )

        for ds_ in pend_scat:
            if ds_ is not None:
                for d in ds_:
                    d.wait()
        plsc.subcore_barrier()

        for j, acc in enumerate((aLx, aLy, aLz, aKx, aKy, aKz)):
            off = (cid * 6 + j) * VP + sid * CHK
            pltpu.sync_copy(acc.at[pl.ds(sid * CHK, CHK)], xb)
            pltpu.sync_copy(xb, out_hbm.at[pl.ds(off, CHK)])

    kfn = pl.kernel(
        body,
        out_type=[jax.ShapeDtypeStruct((NC * 6 * VP,), jnp.float32),
                  jax.ShapeDtypeStruct((3 * V,), jnp.float32)],
        mesh=mesh,
        scratch_types=[
            pltpu.VMEM_SHARED((VP,), jnp.float32),
            pltpu.VMEM_SHARED((VP,), jnp.float32),
            pltpu.VMEM_SHARED((VP,), jnp.float32),
            pltpu.VMEM_SHARED((VP,), jnp.float32),
            pltpu.VMEM_SHARED((VP,), jnp.float32),
            pltpu.VMEM_SHARED((VP,), jnp.float32),
            pltpu.VMEM_SHARED((3 * VP,), jnp.float32),
            pltpu.VMEM((9408,), jnp.float32),
            pltpu.VMEM((CHK,), jnp.float32),
            pltpu.VMEM((3 * LANES,), jnp.float32),
            pltpu.VMEM((S,), jnp.int32),
            pltpu.VMEM((S,), jnp.int32),
            pltpu.VMEM((S,), jnp.float32),
            pltpu.VMEM((S,), jnp.int32),
            pltpu.VMEM((S,), jnp.int32),
            pltpu.VMEM((S,), jnp.float32),
            pltpu.VMEM((S,), jnp.int32),
            pltpu.VMEM((S,), jnp.int32),
            pltpu.VMEM((S,), jnp.int32),
            pltpu.VMEM((S,), jnp.int32),
            pltpu.VMEM((S,), jnp.int32),
            pltpu.VMEM((S,), jnp.int32),
            pltpu.VMEM((S,), jnp.float32),
            pltpu.VMEM((S,), jnp.float32),
            pltpu.VMEM((S,), jnp.float32),
            pltpu.VMEM((S,), jnp.float32),
            pltpu.VMEM((S,), jnp.float32),
            pltpu.VMEM((S,), jnp.float32),
            pltpu.SemaphoreType.DMA,
            pltpu.SemaphoreType.DMA,
            pltpu.SemaphoreType.DMA,
            pltpu.SemaphoreType.DMA,
            pltpu.SemaphoreType.DMA,
            pltpu.SemaphoreType.DMA,
            pltpu.SemaphoreType.DMA,
        ],
    )
    return kfn(vflat, cpat48, ind_l, val_l, ind_k, val_k)


def _tc_reduce_call(parts_flat):
    def body(p_ref, out_ref):
        def comp(j):
            return (p_ref[pl.ds(j * VP, VP)] +
                    p_ref[pl.ds((6 + j) * VP, VP)])
        eps = jnp.float32(1e-12)
        lx, ly, lz = comp(0) + eps, comp(1) + eps, comp(2) + eps
        norm = jnp.sqrt(lx * lx + ly * ly + lz * lz)
        kx, ky, kz = comp(3), comp(4), comp(5)
        ksq = kx * kx + ky * ky + kz * kz
        out_ref[0, 0] = jnp.sum(norm) / jnp.float32(V)
        out_ref[0, 1] = jnp.sum(ksq) / jnp.float32(V)

    return pl.pallas_call(
        body,
        out_specs=pl.BlockSpec(memory_space=pltpu.SMEM),
        out_shape=jax.ShapeDtypeStruct((1, 2), jnp.float32),
    )(parts_flat)


def kernel(vertices, center, faces, L_indices, L_values, K_indices, K_values,
           total_num):
    one_i = jnp.asarray(total_num, dtype=jnp.int32) // 4
    one_f = one_i.astype(jnp.float32)

    vflat = vertices.reshape(3 * V)
    c3 = center.reshape(3)
    cpat48 = jnp.tile(c3, LANES)

    parts, vc3 = _sc_spmv_call(vflat, cpat48, L_indices.reshape(2 * NNZ),
                               L_values, K_indices.reshape(2 * NNZ), K_values)
    losses = _tc_reduce_call(parts)

    vc = vc3.reshape(1, V, 3)
    verts_out = jnp.tile(vc, (4, 1, 1)) * one_f
    faces_out = jnp.tile(faces[None], (4, 1, 1)) * one_i

    laplacian_loss = losses[0, 0]
    hexagon_loss = losses[0, 1]
    zero = jnp.float32(0.0)
    return (verts_out, faces_out, laplacian_loss, hexagon_loss, zero, zero)
